# Initial kernel scaffold; baseline (speedup 1.0000x reference)
import jax
import jax.numpy as jnp
from jax.experimental import pallas as pl


def kernel(x, edge_index, batch_size, batch_index, W1, b1, W2, b2, W3, b3, enc_W, enc_b, dec_W, dec_b):
    raise NotImplementedError("write your pallas kernel here")


# speedup vs baseline: 30.3175x; 30.3175x over previous
import functools

import jax
import jax.numpy as jnp
from jax import lax
from jax.experimental import pallas as pl
from jax.experimental.pallas import tpu as pltpu
from jax.experimental.pallas import tpu_sc as plsc

N = 10000
NP = 10112
E = 320000
NC, NS = 2, 16
NW = NC * NS
EW = 10240
EP = EW * NW
BL = 128
NBLK = EW // BL
KIF = 4
ROWS_W = NBLK
CH = NP // NS
ZR = 160
_STAGES = ((0, 160), (160, 160), (320, 160), (480, 152))


def _mesh():
    return plsc.VectorSubcoreMesh(
        core_axis_name="c", subcore_axis_name="s", num_cores=NC, num_subcores=NS
    )


@functools.partial(
    pl.kernel,
    out_type=jax.ShapeDtypeStruct((NC * NP,), jnp.float32),
    mesh=_mesh(),
    scratch_types=[
        pltpu.VMEM((ROWS_W, BL), jnp.int32),
        pltpu.VMEM((BL,), jnp.float32),
        pltpu.VMEM((640,), jnp.float32),
        pltpu.VMEM_SHARED((NP,), jnp.float32),
    ],
)
def _deg_kernel(dst_hbm, out_hbm, dst_v, ones_v, zb_v, acc):
    cid = lax.axis_index("c")
    sid = lax.axis_index("s")
    wid = sid * NC + cid

    def fill_ones(i, _):
        ones_v[pl.ds(i * 16, 16)] = jnp.ones((16,), jnp.float32)
        return 0

    lax.fori_loop(0, BL // 16, fill_ones, 0)

    def fill_zb(i, _):
        zb_v[pl.ds(i * 16, 16)] = jnp.zeros((16,), jnp.float32)
        return 0

    lax.fori_loop(0, 640 // 16, fill_zb, 0)

    pltpu.sync_copy(zb_v.at[pl.ds(0, CH)], acc.at[pl.ds(sid * CH, CH)])
    plsc.subcore_barrier()

    pltpu.sync_copy(dst_hbm.at[pl.ds(wid * ROWS_W, ROWS_W)], dst_v)

    def body(i, _):
        pltpu.sync_copy(ones_v, acc.at[dst_v.at[i]], add=True)
        return 0

    lax.fori_loop(0, NBLK, body, 0)
    plsc.subcore_barrier()
    pltpu.sync_copy(acc.at[pl.ds(sid * CH, CH)], zb_v.at[pl.ds(0, CH)])
    pltpu.sync_copy(
        zb_v.at[pl.ds(0, CH)], out_hbm.at[pl.ds(cid * NP + sid * CH, CH)]
    )


def _make_prop(F):
    @functools.partial(
        pl.kernel,
        out_type=jax.ShapeDtypeStruct((NC, NP, F), jnp.float32),
        mesh=_mesh(),
        scratch_types=[
            pltpu.VMEM((ROWS_W, BL), jnp.int32),
            pltpu.VMEM((ROWS_W, BL), jnp.int32),
            pltpu.VMEM((KIF, BL, F), jnp.float32),
            pltpu.VMEM((ZR, F), jnp.float32),
            pltpu.VMEM_SHARED((NP, F), jnp.float32),
            pltpu.SemaphoreType.DMA,
        ],
        compiler_params=pltpu.CompilerParams(use_tc_tiling_on_sc=False),
    )
    def _prop(src_hbm, dst_hbm, hs_hbm, out_hbm, src_v, dst_v, rows_v, zb_v, acc, sem):
        cid = lax.axis_index("c")
        sid = lax.axis_index("s")
        wid = sid * NC + cid
        fpl = F // 16

        def fill_zb(i, _):
            zb_v[i // fpl, pl.ds((i % fpl) * 16, 16)] = jnp.zeros((16,), jnp.float32)
            return 0

        lax.fori_loop(0, ZR * fpl, fill_zb, 0)
        for o, sz in _STAGES:
            pltpu.sync_copy(
                zb_v.at[pl.ds(0, sz)], acc.at[pl.ds(sid * CH + o, sz)]
            )
        plsc.subcore_barrier()

        pltpu.sync_copy(src_hbm.at[pl.ds(wid * ROWS_W, ROWS_W)], src_v)
        pltpu.sync_copy(dst_hbm.at[pl.ds(wid * ROWS_W, ROWS_W)], dst_v)

        def body(g, _):
            descs = []
            for j in range(KIF):
                d = pltpu.async_copy(
                    hs_hbm.at[src_v.at[g * KIF + j]], rows_v.at[j], sem
                )
                descs.append(d)
            for j in range(KIF):
                descs[j].wait()
            for j in range(KIF):
                pltpu.sync_copy(rows_v.at[j], acc.at[dst_v.at[g * KIF + j]], add=True)
            return 0

        lax.fori_loop(0, NBLK // KIF, body, 0)
        plsc.subcore_barrier()
        for o, sz in _STAGES:
            pltpu.sync_copy(acc.at[pl.ds(sid * CH + o, sz)], zb_v.at[pl.ds(0, sz)])
            pltpu.sync_copy(
                zb_v.at[pl.ds(0, sz)], out_hbm.at[cid, pl.ds(sid * CH + o, sz)]
            )

    return _prop


_PROP = {f: _make_prop(f) for f in (64, 32, 16)}


_MBLK = NP // 8


def _mm_body(x_ref, w_ref, o_ref):
    o_ref[...] = jnp.dot(x_ref[...], w_ref[...], preferred_element_type=jnp.float32)


def _mm(xp, w):
    kin, kout = w.shape
    return pl.pallas_call(
        _mm_body,
        grid=(8,),
        in_specs=[
            pl.BlockSpec((_MBLK, kin), lambda i: (i, 0)),
            pl.BlockSpec((kin, kout), lambda i: (0, 0)),
        ],
        out_specs=pl.BlockSpec((_MBLK, kout), lambda i: (i, 0)),
        out_shape=jax.ShapeDtypeStruct((NP, kout), jnp.float32),
    )(xp, w)


def _scale_body(h_ref, d_ref, dinv_ref, hs_ref):
    dinv = lax.rsqrt(d_ref[0] + d_ref[1] + 1.0)
    dinv_ref[...] = dinv
    hs_ref[...] = h_ref[...] * dinv


def _scale(h1, degp):
    return pl.pallas_call(
        _scale_body,
        grid=(8,),
        in_specs=[
            pl.BlockSpec((_MBLK, 64), lambda i: (i, 0)),
            pl.BlockSpec((2, _MBLK, 1), lambda i: (0, i, 0)),
        ],
        out_specs=[
            pl.BlockSpec((_MBLK, 1), lambda i: (i, 0)),
            pl.BlockSpec((_MBLK, 64), lambda i: (i, 0)),
        ],
        out_shape=[
            jax.ShapeDtypeStruct((NP, 1), jnp.float32),
            jax.ShapeDtypeStruct((NP, 64), jnp.float32),
        ],
    )(h1, degp)


def _comb_body(p_ref, hs_ref, dinv_ref, b_ref, w_ref, o_ref):
    a = dinv_ref[...] * (p_ref[0] + p_ref[1] + hs_ref[...]) + b_ref[...]
    a = jnp.maximum(a, 0.0)
    o_ref[...] = (
        jnp.dot(a, w_ref[...], preferred_element_type=jnp.float32) * dinv_ref[...]
    )


def _comb_mm(p, hs, dinv, b, w):
    fin, fout = w.shape
    return pl.pallas_call(
        _comb_body,
        grid=(8,),
        in_specs=[
            pl.BlockSpec((2, _MBLK, fin), lambda i: (0, i, 0)),
            pl.BlockSpec((_MBLK, fin), lambda i: (i, 0)),
            pl.BlockSpec((_MBLK, 1), lambda i: (i, 0)),
            pl.BlockSpec((1, fin), lambda i: (0, 0)),
            pl.BlockSpec((fin, fout), lambda i: (0, 0)),
        ],
        out_specs=pl.BlockSpec((_MBLK, fout), lambda i: (i, 0)),
        out_shape=jax.ShapeDtypeStruct((NP, fout), jnp.float32),
    )(p, hs, dinv, b, w)


def _enc_body(p_ref, hs_ref, dinv_ref, b_ref, encp_ref, encb_ref, decw_ref,
              decb_ref, o_ref, zacc):
    h3 = dinv_ref[...] * (p_ref[0] + p_ref[1] + hs_ref[...]) + b_ref[...]
    c = lax.dot_general(
        h3, encp_ref[...], (((0,), (0,)), ((), ())),
        preferred_element_type=jnp.float32,
    )
    z = c[0:1, 0:64]
    for f in range(1, 16):
        z = z + c[f:f + 1, 64 * f:64 * (f + 1)]
    i = pl.program_id(0)

    @pl.when(i == 0)
    def _():
        zacc[...] = z

    @pl.when(i > 0)
    def _():
        zacc[...] = zacc[...] + z

    @pl.when(i == 9)
    def _():
        zf = zacc[...] + encb_ref[...]
        o_ref[...] = (
            jnp.dot(zf, decw_ref[...], preferred_element_type=jnp.float32)
            + decb_ref[...]
        )


def _enc(p3, hs3, dinv, b3, encp, encb, decw, decb):
    return pl.pallas_call(
        _enc_body,
        grid=(10,),
        in_specs=[
            pl.BlockSpec((2, 1000, 16), lambda i: (0, i, 0)),
            pl.BlockSpec((1000, 16), lambda i: (i, 0)),
            pl.BlockSpec((1000, 1), lambda i: (i, 0)),
            pl.BlockSpec((1, 16), lambda i: (0, 0)),
            pl.BlockSpec((1000, 1024), lambda i: (i, 0)),
            pl.BlockSpec((1, 64), lambda i: (0, 0)),
            pl.BlockSpec((64, N), lambda i: (0, 0)),
            pl.BlockSpec((1, N), lambda i: (0, 0)),
        ],
        out_specs=pl.BlockSpec((1, N), lambda i: (0, 0)),
        out_shape=jax.ShapeDtypeStruct((1, N), jnp.float32),
        scratch_shapes=[pltpu.VMEM((1, 64), jnp.float32)],
    )(p3, hs3, dinv, b3, encp, encb, decw, decb)


def kernel(x, edge_index, batch_size, batch_index, W1, b1, W2, b2, W3, b3,
           enc_W, enc_b, dec_W, dec_b):
    src = edge_index[0].astype(jnp.int32)
    dst = edge_index[1].astype(jnp.int32)
    npad = EP - E
    pad = N + (jnp.arange(npad, dtype=jnp.int32) % 64)
    src_p = jnp.concatenate([src, pad]).reshape(EP // BL, BL)
    dst_p = jnp.concatenate([dst, pad]).reshape(EP // BL, BL)
    x_p = jnp.pad(x, ((0, NP - N), (0, 0)))

    degp = _deg_kernel(dst_p)
    h1 = _mm(x_p, W1)
    dinv, hs1 = _scale(h1, degp.reshape(NC, NP, 1))
    p1 = _PROP[64](src_p, dst_p, hs1)
    hs2 = _comb_mm(p1, hs1, dinv, b1.reshape(1, 64), W2)
    p2 = _PROP[32](src_p, dst_p, hs2)
    hs3 = _comb_mm(p2, hs2, dinv, b2.reshape(1, 32), W3)
    p3 = _PROP[16](src_p, dst_p, hs3)

    out = _enc(
        p3, hs3, dinv, b3.reshape(1, 16),
        enc_W.reshape(N, 16 * 64), enc_b.reshape(1, 64),
        dec_W, dec_b.reshape(1, N),
    )
    return out
